# baseline (speedup 1.0000x reference)
import functools

import jax
import jax.numpy as jnp
import numpy as np
from jax import lax
from jax.experimental import pallas as pl
from jax.experimental.pallas import tpu as pltpu
from jax.experimental.pallas import tpu_sc as plsc

FEAT_DIM = 80
DEPTH_CHANNELS = 112
X_BOUND = (-50.0, 50.0, 0.5)
Y_BOUND = (-50.0, 50.0, 0.5)
NX = 200
NY = 200
DEPTH_MIN = 1.0
DEPTH_MAX = 57.0

NW = 32
NVOX = NX * NY
NCELL = 64
CELL = 25
CROWS = CELL * CELL
CHUNK = 1024
VECS = CHUNK // 16
UNROLL = 8
NCAM = 6
HWPIX = 704
CPC = 77
NCHUNKS = NCAM * CPC
MASKPAD = 512


def _cell_slot(x_idx, y_idx):
    cell = (x_idx // CELL) * 8 + y_idx // CELL
    slot = (x_idx % CELL) * CELL + y_idx % CELL
    return cell, slot


def _inverse_perm():
    lin = np.arange(NVOX)
    x, y = lin // NY, lin % NY
    cell = (x // CELL) * 8 + y // CELL
    slot = (x % CELL) * CELL + y % CELL
    return jnp.asarray(cell * CROWS + slot, dtype=jnp.int32)


def _point_meta(intrinsics, extrinsics, feat_h, feat_w, img_h, img_w):
    D = DEPTH_CHANNELS
    depth_bins = jnp.linspace(DEPTH_MIN, DEPTH_MAX, D)
    ys, xs = jnp.meshgrid(jnp.arange(feat_h, dtype=jnp.float32),
                          jnp.arange(feat_w, dtype=jnp.float32), indexing='ij')
    ds = jnp.broadcast_to(depth_bins[:, None, None], (D, feat_h, feat_w))
    xs = jnp.broadcast_to(xs[None], (D, feat_h, feat_w)) * (img_w / feat_w)
    ys = jnp.broadcast_to(ys[None], (D, feat_h, feat_w)) * (img_h / feat_h)
    frustum = jnp.stack([xs, ys, ds], axis=-1)
    pts = frustum.reshape(-1, 3)
    pts = jnp.stack([pts[:, 0] * pts[:, 2], pts[:, 1] * pts[:, 2], pts[:, 2]], axis=-1)
    inv_K = jnp.linalg.inv(intrinsics)
    cam = jnp.einsum('bnij,pj->bnpi', inv_K, pts)
    ones = jnp.ones_like(cam[..., :1])
    cam_h = jnp.concatenate([cam, ones], axis=-1)
    ego = jnp.einsum('bnij,bnpj->bnpi', extrinsics, cam_h)
    geom = ego[..., :3]
    x_idx = ((geom[..., 0] - X_BOUND[0]) / X_BOUND[2]).astype(jnp.int32)
    y_idx = ((geom[..., 1] - Y_BOUND[0]) / Y_BOUND[2]).astype(jnp.int32)
    valid = (x_idx >= 0) & (x_idx < NX) & (y_idx >= 0) & (y_idx < NY)
    x_idx = jnp.where(valid, x_idx, 0).reshape(-1)
    y_idx = jnp.where(valid, y_idx, 0).reshape(-1)
    valid = valid.reshape(-1)
    cell, slot = _cell_slot(x_idx, y_idx)
    cell = jnp.where(valid, cell, NCELL)
    slot = jnp.where(valid, slot, 0)
    P = valid.shape[0]
    pidx = jnp.arange(P, dtype=jnp.int32)
    pix = pidx % HWPIX
    meta = (cell << 20) | (slot << 10) | pix
    one = jnp.int32(1)
    lo = jnp.where(cell < 32, jnp.left_shift(one, cell), 0)
    hi = jnp.where((cell >= 32) & (cell < NCELL),
                   jnp.left_shift(one, cell - 32), 0)
    mlo = lax.reduce(lo.reshape(NCHUNKS, CHUNK), jnp.int32(0), lax.bitwise_or, (1,))
    mhi = lax.reduce(hi.reshape(NCHUNKS, CHUNK), jnp.int32(0), lax.bitwise_or, (1,))
    mlo = jnp.pad(mlo, (0, MASKPAD - NCHUNKS))
    mhi = jnp.pad(mhi, (0, MASKPAD - NCHUNKS))
    return meta, mlo, mhi


def _sc_body(meta_hbm, w_hbm, ctx_hbm, mlo_hbm, mhi_hbm, out_hbm,
             acc, ctxslab, masklo, maskhi, meta_v, w_v, hit_meta, hit_w):
    t = lax.axis_index("s") * 2 + lax.axis_index("c")

    def zero_hits(i, _):
        hit_meta[pl.ds(i * 16, 16)] = jnp.zeros((16,), jnp.int32)
        return 0
    lax.fori_loop(0, (CHUNK + 32) // 16, zero_hits, 0)

    pltpu.sync_copy(mlo_hbm, masklo)
    pltpu.sync_copy(mhi_hbm, maskhi)

    b2 = (t & 56) + (((t & 7) + 4) & 7)

    for ppass in range(2):
        cell = t if ppass == 0 else 32 + b2
        bit = t if ppass == 0 else b2
        maskbuf = masklo if ppass == 0 else maskhi

        def zero_body(i, _):
            acc[pl.ds(i * 16, 16)] = jnp.zeros((16,), jnp.float32)
            return 0
        lax.fori_loop(0, CROWS * FEAT_DIM // 16, zero_body, 0)

        for n in range(NCAM):
            pltpu.sync_copy(ctx_hbm.at[n], ctxslab)

            def chunk_body(cc, _):
                ci = n * CPC + cc
                msk = maskbuf[pl.ds(ci, 16)][0]

                @pl.when(((msk >> bit) & 1) > 0)
                def _():
                    pltpu.sync_copy(meta_hbm.at[pl.ds(ci * CHUNK, CHUNK)], meta_v)
                    pltpu.sync_copy(w_hbm.at[pl.ds(ci * CHUNK, CHUNK)], w_v)

                    def scan_body(u, nh):
                        for k in range(UNROLL):
                            off = (u * UNROLL + k) * 16
                            m = meta_v[pl.ds(off, 16)]
                            own = (m >> 20) == cell
                            plsc.store_compressed(hit_meta.at[pl.ds(nh, 16)],
                                                  m, mask=own)
                            w = w_v[pl.ds(off, 16)]
                            plsc.store_compressed(hit_w.at[pl.ds(nh, 16)],
                                                  w, mask=own)
                            nh = nh + plsc.all_reduce_population_count(own)[0]
                        return nh

                    nh = lax.fori_loop(0, VECS // UNROLL, scan_body, 0)

                    def hit_body(h, _):
                        m = hit_meta[pl.ds(h, 16)][0]
                        wsc = hit_w[pl.ds(h, 16)][0]
                        base = ((m >> 10) & 0x3FF) * FEAT_DIM
                        cbase = (m & 0x3FF) * FEAT_DIM
                        for q in range(FEAT_DIM // 16):
                            plsc.addupdate(
                                acc.at[pl.ds(base + q * 16, 16)],
                                wsc * ctxslab[pl.ds(cbase + q * 16, 16)])
                        return 0

                    lax.fori_loop(0, nh, hit_body, 0)
                return 0

            lax.fori_loop(0, CPC, chunk_body, 0)

        pltpu.sync_copy(acc, out_hbm.at[cell])


def kernel(image_features, depth_dist, context_features, intrinsics, extrinsics, img_h, img_w):
    Bb, Nn, C, Hh, Ww = context_features.shape
    meta, mlo, mhi = _point_meta(intrinsics, extrinsics, Hh, Ww, img_h, img_w)
    w_flat = depth_dist.reshape(-1)
    ctx = jnp.transpose(context_features, (0, 1, 3, 4, 2)).reshape(Nn, Hh * Ww * C)

    mesh = plsc.VectorSubcoreMesh(core_axis_name="c", subcore_axis_name="s")
    sc = functools.partial(
        pl.kernel, _sc_body, mesh=mesh,
        compiler_params=pltpu.CompilerParams(needs_layout_passes=False,
                                             use_tc_tiling_on_sc=False),
        out_type=jax.ShapeDtypeStruct((NCELL, CROWS * FEAT_DIM), jnp.float32),
        scratch_types=[
            pltpu.VMEM((CROWS * FEAT_DIM,), jnp.float32),
            pltpu.VMEM((HWPIX * FEAT_DIM,), jnp.float32),
            pltpu.VMEM((MASKPAD,), jnp.int32),
            pltpu.VMEM((MASKPAD,), jnp.int32),
            pltpu.VMEM((CHUNK,), jnp.int32),
            pltpu.VMEM((CHUNK,), jnp.float32),
            pltpu.VMEM((CHUNK + 32,), jnp.int32),
            pltpu.VMEM((CHUNK + 32,), jnp.float32),
        ],
    )()
    out = sc(meta, w_flat, ctx, mlo, mhi)

    rows = out.reshape(NCELL * CROWS, C)
    bev = rows[_inverse_perm()].reshape(NX, NY, C)
    return jnp.transpose(bev, (2, 0, 1))[None]


# speedup vs baseline: 1.0032x; 1.0032x over previous
import functools

import jax
import jax.numpy as jnp
import numpy as np
from jax import lax
from jax.experimental import pallas as pl
from jax.experimental.pallas import tpu as pltpu
from jax.experimental.pallas import tpu_sc as plsc

FEAT_DIM = 80
DEPTH_CHANNELS = 112
X_BOUND = (-50.0, 50.0, 0.5)
Y_BOUND = (-50.0, 50.0, 0.5)
NX = 200
NY = 200
DEPTH_MIN = 1.0
DEPTH_MAX = 57.0

NW = 32
NVOX = NX * NY
NCELL = 64
CELL = 25
CROWS = CELL * CELL
CHUNK = 1024
VECS = CHUNK // 16
UNROLL = 8
NCAM = 6
HWPIX = 704
CPC = 77
NCHUNKS = NCAM * CPC
MASKPAD = 512


def _cell_slot(x_idx, y_idx):
    cell = (x_idx // CELL) * 8 + y_idx // CELL
    slot = (x_idx % CELL) * CELL + y_idx % CELL
    return cell, slot


def _inverse_perm():
    lin = np.arange(NVOX)
    x, y = lin // NY, lin % NY
    cell = (x // CELL) * 8 + y // CELL
    slot = (x % CELL) * CELL + y % CELL
    return jnp.asarray(cell * CROWS + slot, dtype=jnp.int32)


def _point_meta(intrinsics, extrinsics, feat_h, feat_w, img_h, img_w):
    D = DEPTH_CHANNELS
    depth_bins = jnp.linspace(DEPTH_MIN, DEPTH_MAX, D)
    ys, xs = jnp.meshgrid(jnp.arange(feat_h, dtype=jnp.float32),
                          jnp.arange(feat_w, dtype=jnp.float32), indexing='ij')
    ds = jnp.broadcast_to(depth_bins[:, None, None], (D, feat_h, feat_w))
    xs = jnp.broadcast_to(xs[None], (D, feat_h, feat_w)) * (img_w / feat_w)
    ys = jnp.broadcast_to(ys[None], (D, feat_h, feat_w)) * (img_h / feat_h)
    frustum = jnp.stack([xs, ys, ds], axis=-1)
    pts = frustum.reshape(-1, 3)
    pts = jnp.stack([pts[:, 0] * pts[:, 2], pts[:, 1] * pts[:, 2], pts[:, 2]], axis=-1)
    inv_K = jnp.linalg.inv(intrinsics)
    cam = jnp.einsum('bnij,pj->bnpi', inv_K, pts)
    ones = jnp.ones_like(cam[..., :1])
    cam_h = jnp.concatenate([cam, ones], axis=-1)
    ego = jnp.einsum('bnij,bnpj->bnpi', extrinsics, cam_h)
    geom = ego[..., :3]
    x_idx = ((geom[..., 0] - X_BOUND[0]) / X_BOUND[2]).astype(jnp.int32)
    y_idx = ((geom[..., 1] - Y_BOUND[0]) / Y_BOUND[2]).astype(jnp.int32)
    valid = (x_idx >= 0) & (x_idx < NX) & (y_idx >= 0) & (y_idx < NY)
    x_idx = jnp.where(valid, x_idx, 0).reshape(-1)
    y_idx = jnp.where(valid, y_idx, 0).reshape(-1)
    valid = valid.reshape(-1)
    cell, slot = _cell_slot(x_idx, y_idx)
    cell = jnp.where(valid, cell, NCELL)
    slot = jnp.where(valid, slot, 0)
    P = valid.shape[0]
    pidx = jnp.arange(P, dtype=jnp.int32)
    pix = pidx % HWPIX
    meta = (cell << 20) | (slot << 10) | pix
    one = jnp.int32(1)
    lo = jnp.where(cell < 32, jnp.left_shift(one, cell), 0)
    hi = jnp.where((cell >= 32) & (cell < NCELL),
                   jnp.left_shift(one, cell - 32), 0)
    mlo = lax.reduce(lo.reshape(NCHUNKS, CHUNK), jnp.int32(0), lax.bitwise_or, (1,))
    mhi = lax.reduce(hi.reshape(NCHUNKS, CHUNK), jnp.int32(0), lax.bitwise_or, (1,))
    mlo = jnp.pad(mlo, (0, MASKPAD - NCHUNKS))
    mhi = jnp.pad(mhi, (0, MASKPAD - NCHUNKS))
    return meta, mlo, mhi


def _sc_body(meta_hbm, w_hbm, ctx_hbm, mlo_hbm, mhi_hbm, out_hbm,
             acc, ctxslab, masklo, maskhi, meta_v, w_v, hit_meta, hit_w):
    t = lax.axis_index("s") * 2 + lax.axis_index("c")

    def zero_hits(i, _):
        hit_meta[pl.ds(i * 16, 16)] = jnp.zeros((16,), jnp.int32)
        return 0
    lax.fori_loop(0, (CHUNK + 32) // 16, zero_hits, 0)

    pltpu.sync_copy(mlo_hbm, masklo)
    pltpu.sync_copy(mhi_hbm, maskhi)

    b2 = (t & 56) + (((t & 7) + 4) & 7)

    for ppass in range(2):
        cell = t if ppass == 0 else 32 + b2
        bit = t if ppass == 0 else b2
        maskbuf = masklo if ppass == 0 else maskhi

        def zero_body(i, _):
            acc[pl.ds(i * 16, 16)] = jnp.zeros((16,), jnp.float32)
            return 0
        lax.fori_loop(0, CROWS * FEAT_DIM // 16, zero_body, 0)

        for n in range(NCAM):
            pltpu.sync_copy(ctx_hbm.at[n], ctxslab)

            def chunk_body(cc, _):
                ci = n * CPC + cc
                msk = maskbuf[pl.ds(ci, 16)][0]

                @pl.when(((msk >> bit) & 1) > 0)
                def _():
                    pltpu.sync_copy(meta_hbm.at[pl.ds(ci * CHUNK, CHUNK)], meta_v)
                    pltpu.sync_copy(w_hbm.at[pl.ds(ci * CHUNK, CHUNK)], w_v)

                    def scan_body(u, nh):
                        for k in range(UNROLL):
                            off = (u * UNROLL + k) * 16
                            m = meta_v[pl.ds(off, 16)]
                            own = (m >> 20) == cell
                            plsc.store_compressed(hit_meta.at[pl.ds(nh, 16)],
                                                  m, mask=own)
                            w = w_v[pl.ds(off, 16)]
                            plsc.store_compressed(hit_w.at[pl.ds(nh, 16)],
                                                  w, mask=own)
                            nh = nh + plsc.all_reduce_population_count(own)[0]
                        return nh

                    nh = lax.fori_loop(0, VECS // UNROLL, scan_body, 0)

                    def one_hit(h):
                        m = hit_meta[pl.ds(h, 16)][0]
                        wsc = hit_w[pl.ds(h, 16)][0]
                        base = ((m >> 10) & 0x3FF) * FEAT_DIM
                        cbase = (m & 0x3FF) * FEAT_DIM
                        for q in range(FEAT_DIM // 16):
                            plsc.addupdate(
                                acc.at[pl.ds(base + q * 16, 16)],
                                wsc * ctxslab[pl.ds(cbase + q * 16, 16)])

                    def hit4_body(h4, _):
                        for j in range(4):
                            one_hit(h4 * 4 + j)
                        return 0

                    def hit_body(h, _):
                        one_hit(h)
                        return 0

                    lax.fori_loop(0, nh >> 2, hit4_body, 0)
                    lax.fori_loop(nh & ~3, nh, hit_body, 0)
                return 0

            lax.fori_loop(0, CPC, chunk_body, 0)

        pltpu.sync_copy(acc, out_hbm.at[cell])


def kernel(image_features, depth_dist, context_features, intrinsics, extrinsics, img_h, img_w):
    Bb, Nn, C, Hh, Ww = context_features.shape
    meta, mlo, mhi = _point_meta(intrinsics, extrinsics, Hh, Ww, img_h, img_w)
    w_flat = depth_dist.reshape(-1)
    ctx = jnp.transpose(context_features, (0, 1, 3, 4, 2)).reshape(Nn, Hh * Ww * C)

    mesh = plsc.VectorSubcoreMesh(core_axis_name="c", subcore_axis_name="s")
    sc = functools.partial(
        pl.kernel, _sc_body, mesh=mesh,
        compiler_params=pltpu.CompilerParams(needs_layout_passes=False,
                                             use_tc_tiling_on_sc=False),
        out_type=jax.ShapeDtypeStruct((NCELL, CROWS * FEAT_DIM), jnp.float32),
        scratch_types=[
            pltpu.VMEM((CROWS * FEAT_DIM,), jnp.float32),
            pltpu.VMEM((HWPIX * FEAT_DIM,), jnp.float32),
            pltpu.VMEM((MASKPAD,), jnp.int32),
            pltpu.VMEM((MASKPAD,), jnp.int32),
            pltpu.VMEM((CHUNK,), jnp.int32),
            pltpu.VMEM((CHUNK,), jnp.float32),
            pltpu.VMEM((CHUNK + 32,), jnp.int32),
            pltpu.VMEM((CHUNK + 32,), jnp.float32),
        ],
    )()
    out = sc(meta, w_flat, ctx, mlo, mhi)

    rows = out.reshape(NCELL * CROWS, C)
    bev = rows[_inverse_perm()].reshape(NX, NY, C)
    return jnp.transpose(bev, (2, 0, 1))[None]


# speedup vs baseline: 1.1014x; 1.0979x over previous
import functools

import jax
import jax.numpy as jnp
from jax import lax
from jax.experimental import pallas as pl
from jax.experimental.pallas import tpu as pltpu
from jax.experimental.pallas import tpu_sc as plsc

FEAT_DIM = 80
DEPTH_CHANNELS = 112
X_BOUND = (-50.0, 50.0, 0.5)
Y_BOUND = (-50.0, 50.0, 0.5)
NX = 200
NY = 200
DEPTH_MIN = 1.0
DEPTH_MAX = 57.0

NW = 32
NVOX = NX * NY
ROWS = NVOX // NW
CHUNK = 2048
VECS = CHUNK // 16
UNROLL = 8
GDEPTH = 4


def _point_meta(intrinsics, extrinsics, feat_h, feat_w, img_h, img_w):
    D = DEPTH_CHANNELS
    depth_bins = jnp.linspace(DEPTH_MIN, DEPTH_MAX, D)
    ys, xs = jnp.meshgrid(jnp.arange(feat_h, dtype=jnp.float32),
                          jnp.arange(feat_w, dtype=jnp.float32), indexing='ij')
    ds = jnp.broadcast_to(depth_bins[:, None, None], (D, feat_h, feat_w))
    xs = jnp.broadcast_to(xs[None], (D, feat_h, feat_w)) * (img_w / feat_w)
    ys = jnp.broadcast_to(ys[None], (D, feat_h, feat_w)) * (img_h / feat_h)
    frustum = jnp.stack([xs, ys, ds], axis=-1)
    pts = frustum.reshape(-1, 3)
    pts = jnp.stack([pts[:, 0] * pts[:, 2], pts[:, 1] * pts[:, 2], pts[:, 2]], axis=-1)
    inv_K = jnp.linalg.inv(intrinsics)
    cam = jnp.einsum('bnij,pj->bnpi', inv_K, pts)
    ones = jnp.ones_like(cam[..., :1])
    cam_h = jnp.concatenate([cam, ones], axis=-1)
    ego = jnp.einsum('bnij,bnpj->bnpi', extrinsics, cam_h)
    geom = ego[..., :3]
    x_idx = ((geom[..., 0] - X_BOUND[0]) / X_BOUND[2]).astype(jnp.int32)
    y_idx = ((geom[..., 1] - Y_BOUND[0]) / Y_BOUND[2]).astype(jnp.int32)
    valid = (x_idx >= 0) & (x_idx < NX) & (y_idx >= 0) & (y_idx < NY)
    lin = (x_idx * NY + y_idx).reshape(-1)
    valid = valid.reshape(-1)
    P = lin.shape[0]
    hw = feat_h * feat_w
    pidx = jnp.arange(P, dtype=jnp.int32)
    col = (pidx // (D * hw)) * hw + pidx % hw
    owner = jnp.where(valid, lin & (NW - 1), NW)
    row = jnp.where(valid, lin >> 5, 0)
    return (owner << 24) | (row << 13) | col


def _sc_body(meta_hbm, w_hbm, ctx_hbm, out_hbm,
             acc, meta_v, w_v, hit_meta, hit_w, ctxbuf, msem, wsem, gsem):
    t = lax.axis_index("s") * 2 + lax.axis_index("c")
    nchunks = meta_hbm.shape[0] // CHUNK

    def zero_body(i, _):
        acc[pl.ds(i * 16, 16)] = jnp.zeros((16,), jnp.float32)
        return 0
    lax.fori_loop(0, ROWS * FEAT_DIM // 16, zero_body, 0)

    def zero_hits(i, _):
        hit_meta[pl.ds(i * 16, 16)] = jnp.zeros((16,), jnp.int32)
        return 0
    lax.fori_loop(0, (CHUNK + 32) // 16, zero_hits, 0)

    def chunk_copies(ci, b):
        src = meta_hbm.at[pl.ds(ci * CHUNK, CHUNK)]
        cm = pltpu.make_async_copy(src, meta_v.at[pl.ds(b * CHUNK, CHUNK)], msem.at[b])
        srcw = w_hbm.at[pl.ds(ci * CHUNK, CHUNK)]
        cw = pltpu.make_async_copy(srcw, w_v.at[pl.ds(b * CHUNK, CHUNK)], wsem.at[b])
        return cm, cw

    def issue_chunk(ci, b):
        @pl.when(ci < nchunks)
        def _():
            cm, cw = chunk_copies(ci, b)
            cm.start()
            cw.start()

    issue_chunk(0, 0)

    def chunk_body(ci, _):
        b = ci & 1
        issue_chunk(ci + 1, 1 - b)
        cm, cw = chunk_copies(ci, b)
        cm.wait()
        cw.wait()
        mbase = b * CHUNK

        def scan_body(u, nh):
            for k in range(UNROLL):
                off = mbase + (u * UNROLL + k) * 16
                m = meta_v[pl.ds(off, 16)]
                own = (m >> 24) == t
                plsc.store_compressed(hit_meta.at[pl.ds(nh, 16)], m, mask=own)
                w = w_v[pl.ds(off, 16)]
                plsc.store_compressed(hit_w.at[pl.ds(nh, 16)], w, mask=own)
                nh = nh + plsc.all_reduce_population_count(own)[0]
            return nh

        nh = lax.fori_loop(0, VECS // UNROLL, scan_body, 0)
        ngroups = (nh + 15) >> 4

        def gather_copy(g, gb):
            mv = hit_meta[pl.ds(g * 16, 16)]
            colv = mv & 0x1FFF
            return pltpu.make_async_copy(ctx_hbm.at[colv],
                                         ctxbuf.at[pl.ds(gb * 16, 16)], gsem.at[gb])

        def issue_gather(g, gb):
            @pl.when(g < ngroups)
            def _():
                gather_copy(g, gb).start()

        for pg in range(GDEPTH):
            issue_gather(pg, pg)

        def group_body(g, _):
            gb = g & (GDEPTH - 1)
            gather_copy(g, gb).wait()
            cnt = jnp.minimum(nh - g * 16, 16)
            cbase = gb * 16

            def hit_body(i, _):
                m = hit_meta[pl.ds(g * 16 + i, 16)][0]
                wsc = hit_w[pl.ds(g * 16 + i, 16)][0]
                base = ((m >> 13) & 0x7FF) * FEAT_DIM
                for q in range(FEAT_DIM // 16):
                    plsc.addupdate(acc.at[pl.ds(base + q * 16, 16)],
                                   wsc * ctxbuf[cbase + i, pl.ds(q * 16, 16)])
                return 0

            lax.fori_loop(0, cnt, hit_body, 0)
            issue_gather(g + GDEPTH, gb)
            return 0

        lax.fori_loop(0, ngroups, group_body, 0)
        return 0

    lax.fori_loop(0, nchunks, chunk_body, 0)
    pltpu.sync_copy(acc, out_hbm.at[t])


def kernel(image_features, depth_dist, context_features, intrinsics, extrinsics, img_h, img_w):
    Bb, Nn, C, Hh, Ww = context_features.shape
    meta = _point_meta(intrinsics, extrinsics, Hh, Ww, img_h, img_w)
    w_flat = depth_dist.reshape(-1)
    ctx = jnp.transpose(context_features, (0, 1, 3, 4, 2)).reshape(Nn * Hh * Ww, C)

    mesh = plsc.VectorSubcoreMesh(core_axis_name="c", subcore_axis_name="s")
    sc = functools.partial(
        pl.kernel, _sc_body, mesh=mesh,
        compiler_params=pltpu.CompilerParams(needs_layout_passes=False,
                                             use_tc_tiling_on_sc=False),
        out_type=jax.ShapeDtypeStruct((NW, ROWS * FEAT_DIM), jnp.float32),
        scratch_types=[
            pltpu.VMEM((ROWS * FEAT_DIM,), jnp.float32),
            pltpu.VMEM((2 * CHUNK,), jnp.int32),
            pltpu.VMEM((2 * CHUNK,), jnp.float32),
            pltpu.VMEM((CHUNK + 32,), jnp.int32),
            pltpu.VMEM((CHUNK + 32,), jnp.float32),
            pltpu.VMEM((GDEPTH * 16, FEAT_DIM), jnp.float32),
            pltpu.SemaphoreType.DMA((2,)),
            pltpu.SemaphoreType.DMA((2,)),
            pltpu.SemaphoreType.DMA((GDEPTH,)),
        ],
    )()
    out = sc(meta, w_flat, ctx)

    bev = out.reshape(NW, ROWS, C).transpose(1, 0, 2).reshape(NX, NY, C)
    return jnp.transpose(bev, (2, 0, 1))[None]
